# Initial kernel scaffold; baseline (speedup 1.0000x reference)
import jax
import jax.numpy as jnp
from jax.experimental import pallas as pl


def kernel(price, ctr, user_id, item_id, cate_id, shop_id, hist_item_id, hist_cate_id, hist_price, T_user, T_item, T_cate, T_shop, T_hist_item, T_hist_cate, W_ctr, W_hp, g_price, b_price, g_ctr, b_ctr, g_hp, b_hp):
    raise NotImplementedError("write your pallas kernel here")


# speedup vs baseline: 2.8347x; 2.8347x over previous
import functools

import jax
import jax.numpy as jnp
from jax import lax
from jax.experimental import pallas as pl
from jax.experimental.pallas import tpu as pltpu
from jax.experimental.pallas import tpu_sc as plsc

B = 16384
L = 50
D = 16
EPS = 1e-3

NC = 2
NS = 16
NW = NC * NS
RPW = B // NW
CH = 32
NCHUNK = RPW // CH
HB = 100
NHB = CH * L // HB


def _sc_body(prices, ctrs, hp_pad, hidx_i, hidx_c, sidx_all, t_u, t_i, t_c,
             t_s, t_hi, t_hc, consts, out,
             sidx, hidx, sbuf, hbuf, hpb, prb, crb, hps, cst, obuf, sem):
    wid = lax.axis_index("s") * NC + lax.axis_index("c")
    row0 = wid * RPW
    iota = lax.iota(jnp.int32, 16)
    zeros = jnp.zeros((16,), jnp.float32)

    pltpu.sync_copy(consts, cst)
    for t in range(4):
        pltpu.sync_copy(sidx_all.at[pl.ds(t * 128 + wid * 4, 4)],
                        sidx.at[pl.ds(t * 4, 4)])
    singles = (t_u, t_i, t_c, t_s)
    hs = []
    for t in range(4):
        for k in range(4):
            hs.append(pltpu.async_copy(
                singles[t].at[sidx.at[t * 4 + k]],
                sbuf.at[pl.ds(t * 512 + k * 128, 128)], sem))
    for h in hs:
        h.wait()

    pscale = cst[0, :]
    pofs = cst[1, :]
    wce = cst[2, :]
    bce = cst[3, :]
    whp = cst[4, :]
    bhp = cst[5, :]

    def chunk_body(ch, _):
        base = row0 + ch * CH
        stage = [
            pltpu.async_copy(hidx_i.at[pl.ds(wid * 256 + ch * 16, 16)],
                             hidx.at[pl.ds(0, 16)], sem),
            pltpu.async_copy(hidx_c.at[pl.ds(wid * 256 + ch * 16, 16)],
                             hidx.at[pl.ds(16, 16)], sem),
            pltpu.async_copy(hp_pad.at[pl.ds(base * 64, CH * 64)], hpb, sem),
            pltpu.async_copy(prices.at[pl.ds(base, CH)], prb, sem),
            pltpu.async_copy(ctrs.at[pl.ds(base, CH)], crb, sem),
        ]
        for h in stage:
            h.wait()
        gathers = []
        for j in range(NHB):
            gathers.append(pltpu.async_copy(
                t_hi.at[hidx.at[j]], hbuf.at[pl.ds(j * HB, HB)], sem))
        for j in range(NHB):
            gathers.append(pltpu.async_copy(
                t_hc.at[hidx.at[16 + j]],
                hbuf.at[pl.ds(CH * L + j * HB, HB)], sem))
        for h in gathers:
            h.wait()

        for half in range(2):
            rows = half * 16 + iota
            pv = prb[pl.ds(half * 16, 16)]
            plsc.store_scatter(obuf, [rows * 129], pv * pscale + pofs)
            acc = zeros
            for j in range(64):
                acc = acc + plsc.load_gather(hpb, [rows * 64 + j])
            hps[pl.ds(half * 16, 16)] = acc

        def row_body(r, _):
            rbase = iota * 0 + r * 129
            cb = plsc.load_gather(crb, [iota * 0 + r])
            plsc.store_scatter(obuf, [rbase + 1 + iota], cb * wce + bce)
            for t in range(4):
                v = sbuf[t * 512 + ch * CH + r, :]
                plsc.store_scatter(obuf, [rbase + 17 + 16 * t + iota], v)
            hv = plsc.load_gather(hps, [iota * 0 + r])
            plsc.store_scatter(obuf, [rbase + 81 + iota], hv * whp + bhp)
            acc_i = zeros
            for j in range(L):
                acc_i = acc_i + hbuf[r * L + j, :]
            plsc.store_scatter(obuf, [rbase + 97 + iota], acc_i * (1.0 / L))
            acc_c = zeros
            for j in range(L):
                acc_c = acc_c + hbuf[CH * L + r * L + j, :]
            plsc.store_scatter(obuf, [rbase + 113 + iota], acc_c * (1.0 / L))
            return 0

        lax.fori_loop(0, CH, row_body, 0)
        pltpu.sync_copy(obuf, out.at[pl.ds(base * 129, CH * 129)])
        return 0

    lax.fori_loop(0, NCHUNK, chunk_body, 0)


@functools.partial(
    pl.kernel,
    out_type=jax.ShapeDtypeStruct((B * 129,), jnp.float32),
    mesh=plsc.VectorSubcoreMesh(core_axis_name="c", subcore_axis_name="s",
                                num_cores=NC),
    compiler_params=pltpu.CompilerParams(needs_layout_passes=False,
                                         use_tc_tiling_on_sc=False),
    scratch_types=[
        pltpu.VMEM((16, 128), jnp.int32),
        pltpu.VMEM((32, HB), jnp.int32),
        pltpu.VMEM((4 * RPW, D), jnp.float32),
        pltpu.VMEM((2 * CH * L, D), jnp.float32),
        pltpu.VMEM((CH * 64,), jnp.float32),
        pltpu.VMEM((CH,), jnp.float32),
        pltpu.VMEM((CH,), jnp.float32),
        pltpu.VMEM((CH,), jnp.float32),
        pltpu.VMEM((6, 16), jnp.float32),
        pltpu.VMEM((CH * 129,), jnp.float32),
        pltpu.SemaphoreType.DMA,
    ],
)
def _sc_kernel(*args):
    _sc_body(*args)


def kernel(price, ctr, user_id, item_id, cate_id, shop_id, hist_item_id,
           hist_cate_id, hist_price, T_user, T_item, T_cate, T_shop,
           T_hist_item, T_hist_cate, W_ctr, W_hp,
           g_price, b_price, g_ctr, b_ctr, g_hp, b_hp):
    rs = 1.0 / jnp.sqrt(jnp.float32(1.0 + EPS))
    consts = jnp.stack([
        jnp.broadcast_to(g_price[0] * rs, (16,)),
        jnp.broadcast_to(b_price[0], (16,)),
        (g_ctr[0] * rs) * W_ctr[0],
        b_ctr[0] * W_ctr[0],
        (g_hp[0] * rs / L) * W_hp[0],
        b_hp[0] * W_hp[0],
    ]).astype(jnp.float32)
    hp_pad = jnp.pad(hist_price, ((0, 0), (0, 64 - L))).reshape(B * 64)
    hidx_i = hist_item_id.reshape(B * L // HB, HB)
    hidx_c = hist_cate_id.reshape(B * L // HB, HB)
    sidx_all = jnp.concatenate([
        user_id.reshape(-1, 128), item_id.reshape(-1, 128),
        cate_id.reshape(-1, 128), shop_id.reshape(-1, 128)], axis=0)
    flat = _sc_kernel(price[:, 0], ctr[:, 0], hp_pad, hidx_i, hidx_c,
                      sidx_all, T_user, T_item, T_cate, T_shop,
                      T_hist_item, T_hist_cate, consts)
    return flat.reshape(B, 129)
